# baseline (speedup 1.0000x reference)
import functools

import jax
import jax.numpy as jnp
from jax import lax
from jax.experimental import pallas as pl
from jax.experimental.pallas import tpu as pltpu
from jax.experimental.pallas import tpu_sc as plsc

N_NODES = 34200
E_TOTAL = 615600
N_EW = 342
D = 128

PN = 34304
CHUNK = 8576
RPT = 536
DUMP = CHUNK
PE = 622592
TPT = 38912
BLKE = 2048
NBLK = TPT // BLKE
BATCH = 96
BR = BATCH // 16
CAP = BLKE + BATCH
NPASS = 2

TC_ROWS = 2144


def _tc_body(x_ref, wr_ref, wt_ref, b_ref, y_ref, base_ref):
    xb = x_ref[...]
    y_ref[...] = jnp.dot(
        xb, wr_ref[...], preferred_element_type=jnp.float32
    ).astype(jnp.bfloat16)
    base_ref[...] = (
        jnp.dot(xb, wt_ref[...], preferred_element_type=jnp.float32) + b_ref[...]
    )


def _tc_matmuls(x_pad, wr_t, wt_t, b2):
    grid = PN // TC_ROWS
    return pl.pallas_call(
        _tc_body,
        grid=(grid,),
        in_specs=[
            pl.BlockSpec((TC_ROWS, D), lambda i: (i, 0)),
            pl.BlockSpec((D, D), lambda i: (0, 0)),
            pl.BlockSpec((D, D), lambda i: (0, 0)),
            pl.BlockSpec((1, D), lambda i: (0, 0)),
        ],
        out_specs=[
            pl.BlockSpec((TC_ROWS, D), lambda i: (i, 0)),
            pl.BlockSpec((TC_ROWS, D), lambda i: (i, 0)),
        ],
        out_shape=[
            jax.ShapeDtypeStruct((PN, D), jnp.bfloat16),
            jax.ShapeDtypeStruct((PN, D), jnp.float32),
        ],
    )(x_pad, wr_t, wt_t, b2)


def _sc_scatter_body(ei, y, base, wts, out,
                     srcb, dstb, gsrc, gdst, gw,
                     gidx0, gidx1, gidx2, gidx3,
                     gbuf0, gbuf1, gbuf2, gbuf3,
                     sidx0, sidx1, sbuf0, sbuf1, wtab, shared,
                     gsem0, gsem1, gsem2, gsem3, ssem0, ssem1):
    cid = lax.axis_index("c")
    sid = lax.axis_index("s")
    iota16 = lax.iota(jnp.int32, 16)

    pltpu.sync_copy(wts, wtab)

    for p in range(NPASS):
        chunk_id = 2 * p + cid
        lo = chunk_id * CHUNK

        plsc.subcore_barrier()
        pltpu.sync_copy(base.at[pl.ds(lo + sid * RPT, RPT)],
                        shared.at[pl.ds(sid * RPT, RPT)])
        plsc.subcore_barrier()

        @pl.loop(0, NBLK)
        def blkloop(blk, lo=lo):
            ebase = sid * TPT + blk * BLKE
            pltpu.sync_copy(ei.at[0, pl.ds(ebase, BLKE)], srcb)
            pltpu.sync_copy(ei.at[1, pl.ds(ebase, BLKE)], dstb)

            @pl.loop(0, BLKE // 16, init_carry=jnp.int32(0), unroll=4)
            def cloop(j, off, ebase=ebase, lo=lo):
                s = srcb[pl.ds(j * 16, 16)]
                d = dstb[pl.ds(j * 16, 16)]
                m = (d >= lo) & (d < lo + CHUNK)
                rel = jnp.where(m, d - lo, DUMP)
                g0 = lax.rem(ebase + j * 16, N_EW)
                lane = g0 + iota16
                lane = jnp.where(lane >= N_EW, lane - N_EW, lane)
                w = plsc.load_gather(wtab, [lane])
                plsc.store_compressed(gsrc.at[pl.ds(off, 16)], s, mask=m)
                plsc.store_compressed(gdst.at[pl.ds(off, 16)], rel, mask=m)
                plsc.store_compressed(gw.at[pl.ds(off, 16)], w, mask=m)
                cnt = plsc.all_reduce_population_count(m)
                return off + jnp.max(cnt)

            n = cloop

            zi = jnp.zeros((16,), jnp.int32)
            dmp = jnp.full((16,), DUMP, jnp.int32)
            zf = jnp.zeros((16,), jnp.float32)
            for t in range(BR):
                gsrc[pl.ds(n + t * 16, 16)] = zi
                gdst[pl.ds(n + t * 16, 16)] = dmp
                gw[pl.ds(n + t * 16, 16)] = zf

            nb = (n + BATCH - 1) // BATCH

            G = ((gidx0, gbuf0, gsem0), (gidx1, gbuf1, gsem1),
                 (gidx2, gbuf2, gsem2), (gidx3, gbuf3, gsem3))
            S = ((sidx0, sbuf0, ssem0), (sidx1, sbuf1, ssem1))

            def stage_fire(b, k):
                gi, gb, gs = G[k]
                for t in range(BR):
                    gi[pl.ds(t * 16, 16)] = gsrc[pl.ds(b * BATCH + t * 16, 16)]
                pltpu.async_copy(y.at[gi], gb, gs)

            def wait_g(k):
                gi, gb, gs = G[k]
                pltpu.make_async_copy(y.at[gi], gb, gs).wait()

            def fire_s(b, m):
                si, sb, ss = S[m]
                for t in range(BR):
                    si[pl.ds(t * 16, 16)] = gdst[pl.ds(b * BATCH + t * 16, 16)]
                pltpu.async_copy(sb, shared.at[si], ss, add=True)

            def wait_s(m):
                si, sb, ss = S[m]
                pltpu.make_async_copy(sb, shared.at[si], ss).wait()

            def scale(b, k, m):
                gb = G[k][1]
                sb = S[m][1]

                @pl.loop(0, BATCH, unroll=8)
                def sloop(j):
                    wv = plsc.load_gather(
                        gw, [jnp.full((16,), b * BATCH + j, jnp.int32)])
                    for t2 in range(4):
                        v = plsc.bitcast(
                            gb[j, pl.ds(t2 * 16, 16)], jnp.bfloat16)
                        a, c = plsc.unpack(
                            v, format=plsc.PackFormat.INTERLEAVED)
                        sb[j, pl.ds(t2 * 16, 16)] = a * wv
                        sb[j, pl.ds(64 + t2 * 16, 16)] = c * wv

            for j in range(4):
                @pl.when(j < nb)
                def _(j=j):
                    stage_fire(j, j)

            @pl.loop(0, (nb + 3) // 4)
            def ploop(i):
                for u in range(4):
                    b = 4 * i + u

                    @pl.when(b < nb)
                    def _(b=b, k=u, m=u % 2):
                        wait_g(k)

                        @pl.when(b >= 2)
                        def _():
                            wait_s(m)

                        scale(b, k, m)

                        @pl.when(b + 4 < nb)
                        def _():
                            stage_fire(b + 4, k)

                        fire_s(b, m)

            po = lax.rem(nb - 1, 2)

            @pl.when((nb >= 1) & (po == 0))
            def _():
                wait_s(0)

            @pl.when((nb >= 1) & (po == 1))
            def _():
                wait_s(1)

            @pl.when((nb >= 2) & (po == 1))
            def _():
                wait_s(0)

            @pl.when((nb >= 2) & (po == 0))
            def _():
                wait_s(1)

        plsc.subcore_barrier()
        pltpu.sync_copy(shared.at[pl.ds(sid * RPT, RPT)],
                        out.at[pl.ds(lo + sid * RPT, RPT)])


@functools.cache
def _get_sc_scatter():
    return pl.kernel(
        _sc_scatter_body,
        out_type=jax.ShapeDtypeStruct((PN, D), jnp.float32),
        mesh=plsc.VectorSubcoreMesh(core_axis_name="c", subcore_axis_name="s"),
        compiler_params=pltpu.CompilerParams(
            needs_layout_passes=False, use_tc_tiling_on_sc=False),
        scratch_types=[
        pltpu.VMEM((BLKE,), jnp.int32),
        pltpu.VMEM((BLKE,), jnp.int32),
        pltpu.VMEM((CAP,), jnp.int32),
        pltpu.VMEM((CAP,), jnp.int32),
        pltpu.VMEM((CAP,), jnp.float32),
        pltpu.VMEM((BATCH,), jnp.int32),
        pltpu.VMEM((BATCH,), jnp.int32),
        pltpu.VMEM((BATCH,), jnp.int32),
        pltpu.VMEM((BATCH,), jnp.int32),
        pltpu.VMEM((BATCH, D // 2), jnp.int32),
        pltpu.VMEM((BATCH, D // 2), jnp.int32),
        pltpu.VMEM((BATCH, D // 2), jnp.int32),
        pltpu.VMEM((BATCH, D // 2), jnp.int32),
        pltpu.VMEM((BATCH,), jnp.int32),
        pltpu.VMEM((BATCH,), jnp.int32),
        pltpu.VMEM((BATCH, D), jnp.float32),
        pltpu.VMEM((BATCH, D), jnp.float32),
            pltpu.VMEM((352,), jnp.float32),
            pltpu.VMEM_SHARED((CHUNK + 8, D), jnp.float32),
            pltpu.SemaphoreType.DMA,
            pltpu.SemaphoreType.DMA,
            pltpu.SemaphoreType.DMA,
            pltpu.SemaphoreType.DMA,
            pltpu.SemaphoreType.DMA,
            pltpu.SemaphoreType.DMA,
        ],
    )


@jax.jit
def kernel(x, edge_index, edge_weights, W_rel, b_rel, W_root):
    x_pad = jnp.zeros((PN, D), jnp.float32).at[:N_NODES].set(x)
    perm = jnp.stack(
        [jnp.arange(64, dtype=jnp.int32),
         jnp.arange(64, dtype=jnp.int32) + 64], axis=1).reshape(-1)
    y_pad, base_pad = _tc_matmuls(
        x_pad, W_rel.T[:, perm], W_root.T, b_rel.reshape(1, D))
    y32 = lax.bitcast_convert_type(
        y_pad.reshape(PN, D // 2, 2), jnp.int32)

    ei = edge_index.astype(jnp.int32)
    pad = PE - E_TOTAL
    ei_pad = jnp.concatenate(
        [ei,
         jnp.stack([jnp.zeros((pad,), jnp.int32),
                    jnp.full((pad,), 2 * N_NODES, jnp.int32)])],
        axis=1)

    w_pad = jnp.zeros((352,), jnp.float32).at[:N_EW].set(edge_weights)

    out_pad = _get_sc_scatter()(ei_pad, y32, base_pad, w_pad)
    return out_pad[:N_NODES]


# speedup vs baseline: 1.5665x; 1.5665x over previous
import functools

import jax
import jax.numpy as jnp
from jax import lax
from jax.experimental import pallas as pl
from jax.experimental.pallas import tpu as pltpu
from jax.experimental.pallas import tpu_sc as plsc

N_NODES = 34200
E_TOTAL = 615600
N_EW = 342
D = 128

PN = 34304
CHUNK = 8576
RPT = 536
DUMP = CHUNK
PE = 622592
TPT = 38912
BLKE = 2048
NBLK = TPT // BLKE
BATCH = 64
BR = BATCH // 16
CAP = BLKE + BATCH
NPASS = 2

TC_ROWS = 2144


def _tc_body(x_ref, wr_ref, wt_ref, b_ref, y_ref, base_ref):
    xb = x_ref[...]
    y_ref[...] = jnp.dot(
        xb, wr_ref[...], preferred_element_type=jnp.float32
    ).astype(jnp.bfloat16)
    base_ref[...] = (
        jnp.dot(xb, wt_ref[...], preferred_element_type=jnp.float32) + b_ref[...]
    )


def _tc_matmuls(x_pad, wr_t, wt_t, b2):
    grid = PN // TC_ROWS
    return pl.pallas_call(
        _tc_body,
        grid=(grid,),
        in_specs=[
            pl.BlockSpec((TC_ROWS, D), lambda i: (i, 0)),
            pl.BlockSpec((D, D), lambda i: (0, 0)),
            pl.BlockSpec((D, D), lambda i: (0, 0)),
            pl.BlockSpec((1, D), lambda i: (0, 0)),
        ],
        out_specs=[
            pl.BlockSpec((TC_ROWS, D), lambda i: (i, 0)),
            pl.BlockSpec((TC_ROWS, D), lambda i: (i, 0)),
        ],
        out_shape=[
            jax.ShapeDtypeStruct((PN, D), jnp.bfloat16),
            jax.ShapeDtypeStruct((PN, D), jnp.float32),
        ],
    )(x_pad, wr_t, wt_t, b2)


def _sc_scatter_body(ei, y, base, wts, out,
                     srcb, dstb, gsrc, gdst, gw,
                     gidx0, gidx1, gidx2, gidx3,
                     gbuf0, gbuf1, gbuf2, gbuf3,
                     sidx0, sidx1, sbuf0, sbuf1, wtab, shared,
                     gsem0, gsem1, gsem2, gsem3, ssem0, ssem1):
    cid = lax.axis_index("c")
    sid = lax.axis_index("s")
    iota16 = lax.iota(jnp.int32, 16)

    pltpu.sync_copy(wts, wtab)

    for p in range(NPASS):
        chunk_id = 2 * p + cid
        lo = chunk_id * CHUNK

        plsc.subcore_barrier()
        pltpu.sync_copy(base.at[pl.ds(lo + sid * RPT, RPT)],
                        shared.at[pl.ds(sid * RPT, RPT)])
        plsc.subcore_barrier()

        @pl.loop(0, NBLK)
        def blkloop(blk, lo=lo):
            ebase = sid * TPT + blk * BLKE
            pltpu.sync_copy(ei.at[0, pl.ds(ebase, BLKE)], srcb)
            pltpu.sync_copy(ei.at[1, pl.ds(ebase, BLKE)], dstb)

            @pl.loop(0, BLKE // 16, init_carry=jnp.int32(0), unroll=4)
            def cloop(j, off, ebase=ebase, lo=lo):
                s = srcb[pl.ds(j * 16, 16)]
                d = dstb[pl.ds(j * 16, 16)]
                m = (d >= lo) & (d < lo + CHUNK)
                rel = jnp.where(m, d - lo, DUMP)
                g0 = lax.rem(ebase + j * 16, N_EW)
                lane = g0 + iota16
                lane = jnp.where(lane >= N_EW, lane - N_EW, lane)
                w = plsc.load_gather(wtab, [lane])
                plsc.store_compressed(gsrc.at[pl.ds(off, 16)], s, mask=m)
                plsc.store_compressed(gdst.at[pl.ds(off, 16)], rel, mask=m)
                plsc.store_compressed(gw.at[pl.ds(off, 16)], w, mask=m)
                cnt = plsc.all_reduce_population_count(m)
                return off + jnp.max(cnt)

            n = cloop

            zi = jnp.zeros((16,), jnp.int32)
            dmp = jnp.full((16,), DUMP, jnp.int32)
            zf = jnp.zeros((16,), jnp.float32)
            for t in range(BR):
                gsrc[pl.ds(n + t * 16, 16)] = zi
                gdst[pl.ds(n + t * 16, 16)] = dmp
                gw[pl.ds(n + t * 16, 16)] = zf

            nb = (n + BATCH - 1) // BATCH

            G = ((gidx0, gbuf0, gsem0), (gidx1, gbuf1, gsem1),
                 (gidx2, gbuf2, gsem2), (gidx3, gbuf3, gsem3))
            S = ((sidx0, sbuf0, ssem0), (sidx1, sbuf1, ssem1))

            def stage_fire(b, k):
                gi, gb, gs = G[k]
                for t in range(BR):
                    gi[pl.ds(t * 16, 16)] = gsrc[pl.ds(b * BATCH + t * 16, 16)]
                pltpu.async_copy(y.at[gi], gb, gs)

            def wait_g(k):
                gi, gb, gs = G[k]
                pltpu.make_async_copy(y.at[gi], gb, gs).wait()

            def fire_s(b, m):
                si, sb, ss = S[m]
                for t in range(BR):
                    si[pl.ds(t * 16, 16)] = gdst[pl.ds(b * BATCH + t * 16, 16)]
                pltpu.async_copy(sb, shared.at[si], ss, add=True)

            def wait_s(m):
                si, sb, ss = S[m]
                pltpu.make_async_copy(sb, shared.at[si], ss).wait()

            def scale(b, k, m):
                gb = G[k][1]
                sb = S[m][1]

                @pl.loop(0, BATCH, unroll=8)
                def sloop(j):
                    wv = plsc.load_gather(
                        gw, [jnp.full((16,), b * BATCH + j, jnp.int32)])
                    for t2 in range(4):
                        v = plsc.bitcast(
                            gb[j, pl.ds(t2 * 16, 16)], jnp.bfloat16)
                        a, c = plsc.unpack(
                            v, format=plsc.PackFormat.INTERLEAVED)
                        sb[j, pl.ds(t2 * 16, 16)] = a * wv
                        sb[j, pl.ds(64 + t2 * 16, 16)] = c * wv

            for j in range(4):
                @pl.when(j < nb)
                def _(j=j):
                    stage_fire(j, j)

            @pl.loop(0, (nb + 3) // 4)
            def ploop(i):
                for u in range(4):
                    b = 4 * i + u

                    @pl.when(b < nb)
                    def _(b=b, k=u, m=u % 2):
                        wait_g(k)

                        @pl.when(b >= 2)
                        def _():
                            wait_s(m)

                        scale(b, k, m)

                        @pl.when(b + 4 < nb)
                        def _():
                            stage_fire(b + 4, k)

                        fire_s(b, m)

            po = lax.rem(nb - 1, 2)

            @pl.when((nb >= 1) & (po == 0))
            def _():
                wait_s(0)

            @pl.when((nb >= 1) & (po == 1))
            def _():
                wait_s(1)

            @pl.when((nb >= 2) & (po == 1))
            def _():
                wait_s(0)

            @pl.when((nb >= 2) & (po == 0))
            def _():
                wait_s(1)

        plsc.subcore_barrier()
        pltpu.sync_copy(shared.at[pl.ds(sid * RPT, RPT)],
                        out.at[pl.ds(lo + sid * RPT, RPT)])


@functools.cache
def _get_sc_scatter():
    return pl.kernel(
        _sc_scatter_body,
        out_type=jax.ShapeDtypeStruct((PN, D), jnp.float32),
        mesh=plsc.VectorSubcoreMesh(core_axis_name="c", subcore_axis_name="s"),
        compiler_params=pltpu.CompilerParams(
            needs_layout_passes=False, use_tc_tiling_on_sc=False),
        scratch_types=[
        pltpu.VMEM((BLKE,), jnp.int32),
        pltpu.VMEM((BLKE,), jnp.int32),
        pltpu.VMEM((CAP,), jnp.int32),
        pltpu.VMEM((CAP,), jnp.int32),
        pltpu.VMEM((CAP,), jnp.float32),
        pltpu.VMEM((BATCH,), jnp.int32),
        pltpu.VMEM((BATCH,), jnp.int32),
        pltpu.VMEM((BATCH,), jnp.int32),
        pltpu.VMEM((BATCH,), jnp.int32),
        pltpu.VMEM((BATCH, D // 2), jnp.int32),
        pltpu.VMEM((BATCH, D // 2), jnp.int32),
        pltpu.VMEM((BATCH, D // 2), jnp.int32),
        pltpu.VMEM((BATCH, D // 2), jnp.int32),
        pltpu.VMEM((BATCH,), jnp.int32),
        pltpu.VMEM((BATCH,), jnp.int32),
        pltpu.VMEM((BATCH, D), jnp.float32),
        pltpu.VMEM((BATCH, D), jnp.float32),
            pltpu.VMEM((352,), jnp.float32),
            pltpu.VMEM_SHARED((CHUNK + 8, D), jnp.float32),
            pltpu.SemaphoreType.DMA,
            pltpu.SemaphoreType.DMA,
            pltpu.SemaphoreType.DMA,
            pltpu.SemaphoreType.DMA,
            pltpu.SemaphoreType.DMA,
            pltpu.SemaphoreType.DMA,
        ],
    )


@jax.jit
def kernel(x, edge_index, edge_weights, W_rel, b_rel, W_root):
    x_pad = jnp.zeros((PN, D), jnp.float32).at[:N_NODES].set(x)
    perm = jnp.stack(
        [jnp.arange(64, dtype=jnp.int32),
         jnp.arange(64, dtype=jnp.int32) + 64], axis=1).reshape(-1)
    y_pad, base_pad = _tc_matmuls(
        x_pad, W_rel.T[:, perm], W_root.T, b_rel.reshape(1, D))
    y32 = lax.bitcast_convert_type(
        y_pad.reshape(PN, D // 2, 2), jnp.int32)

    ei = edge_index.astype(jnp.int32)
    pad = PE - E_TOTAL
    ei_pad = jnp.concatenate(
        [ei,
         jnp.stack([jnp.zeros((pad,), jnp.int32),
                    jnp.full((pad,), 2 * N_NODES, jnp.int32)])],
        axis=1)

    w_pad = jnp.zeros((352,), jnp.float32).at[:N_EW].set(edge_weights)

    out_pad = _get_sc_scatter()(ei_pad, y32, base_pad, w_pad)
    return out_pad[:N_NODES]


# speedup vs baseline: 1.8570x; 1.1854x over previous
import functools

import jax
import jax.numpy as jnp
from jax import lax
from jax.experimental import pallas as pl
from jax.experimental.pallas import tpu as pltpu
from jax.experimental.pallas import tpu_sc as plsc

N_NODES = 34200
E_TOTAL = 615600
N_EW = 342
D = 128

PN = 34304
CHUNK = 8576
RPT = 536
DUMP = CHUNK
PE = 622592
TPT = 38912
BLKE = 2048
NBLK = TPT // BLKE
BATCH = 32
BR = BATCH // 16
CAP = BLKE + BATCH
NPASS = 2

TC_ROWS = 2144


def _tc_body(x_ref, wr_ref, wt_ref, b_ref, y_ref, base_ref):
    xb = x_ref[...]
    y_ref[...] = jnp.dot(
        xb, wr_ref[...], preferred_element_type=jnp.float32
    ).astype(jnp.bfloat16)
    base_ref[...] = (
        jnp.dot(xb, wt_ref[...], preferred_element_type=jnp.float32) + b_ref[...]
    )


def _tc_matmuls(x_pad, wr_t, wt_t, b2):
    grid = PN // TC_ROWS
    return pl.pallas_call(
        _tc_body,
        grid=(grid,),
        in_specs=[
            pl.BlockSpec((TC_ROWS, D), lambda i: (i, 0)),
            pl.BlockSpec((D, D), lambda i: (0, 0)),
            pl.BlockSpec((D, D), lambda i: (0, 0)),
            pl.BlockSpec((1, D), lambda i: (0, 0)),
        ],
        out_specs=[
            pl.BlockSpec((TC_ROWS, D), lambda i: (i, 0)),
            pl.BlockSpec((TC_ROWS, D), lambda i: (i, 0)),
        ],
        out_shape=[
            jax.ShapeDtypeStruct((PN, D), jnp.bfloat16),
            jax.ShapeDtypeStruct((PN, D), jnp.float32),
        ],
    )(x_pad, wr_t, wt_t, b2)


def _sc_scatter_body(ei, y, base, wts, out,
                     srcb, dstb, gsrc, gdst, gw,
                     gidx0, gidx1, gidx2, gidx3,
                     gbuf0, gbuf1, gbuf2, gbuf3,
                     sidx0, sidx1, sbuf0, sbuf1, wtab, shared,
                     gsem0, gsem1, gsem2, gsem3, ssem0, ssem1):
    cid = lax.axis_index("c")
    sid = lax.axis_index("s")
    iota16 = lax.iota(jnp.int32, 16)

    pltpu.sync_copy(wts, wtab)

    for p in range(NPASS):
        chunk_id = 2 * p + cid
        lo = chunk_id * CHUNK

        plsc.subcore_barrier()
        pltpu.sync_copy(base.at[pl.ds(lo + sid * RPT, RPT)],
                        shared.at[pl.ds(sid * RPT, RPT)])
        plsc.subcore_barrier()

        @pl.loop(0, NBLK)
        def blkloop(blk, lo=lo):
            ebase = sid * TPT + blk * BLKE
            pltpu.sync_copy(ei.at[0, pl.ds(ebase, BLKE)], srcb)
            pltpu.sync_copy(ei.at[1, pl.ds(ebase, BLKE)], dstb)

            @pl.loop(0, BLKE // 16, init_carry=jnp.int32(0), unroll=4)
            def cloop(j, off, ebase=ebase, lo=lo):
                s = srcb[pl.ds(j * 16, 16)]
                d = dstb[pl.ds(j * 16, 16)]
                m = (d >= lo) & (d < lo + CHUNK)
                rel = jnp.where(m, d - lo, DUMP)
                g0 = lax.rem(ebase + j * 16, N_EW)
                lane = g0 + iota16
                lane = jnp.where(lane >= N_EW, lane - N_EW, lane)
                w = plsc.load_gather(wtab, [lane])
                plsc.store_compressed(gsrc.at[pl.ds(off, 16)], s, mask=m)
                plsc.store_compressed(gdst.at[pl.ds(off, 16)], rel, mask=m)
                plsc.store_compressed(gw.at[pl.ds(off, 16)], w, mask=m)
                cnt = plsc.all_reduce_population_count(m)
                return off + jnp.max(cnt)

            n = cloop

            zi = jnp.zeros((16,), jnp.int32)
            dmp = jnp.full((16,), DUMP, jnp.int32)
            zf = jnp.zeros((16,), jnp.float32)
            for t in range(BR):
                gsrc[pl.ds(n + t * 16, 16)] = zi
                gdst[pl.ds(n + t * 16, 16)] = dmp
                gw[pl.ds(n + t * 16, 16)] = zf

            nb = (n + BATCH - 1) // BATCH

            G = ((gidx0, gbuf0, gsem0), (gidx1, gbuf1, gsem1),
                 (gidx2, gbuf2, gsem2), (gidx3, gbuf3, gsem3))
            S = ((sidx0, sbuf0, ssem0), (sidx1, sbuf1, ssem1))

            def stage_fire(b, k):
                gi, gb, gs = G[k]
                for t in range(BR):
                    gi[pl.ds(t * 16, 16)] = gsrc[pl.ds(b * BATCH + t * 16, 16)]
                pltpu.async_copy(y.at[gi], gb, gs)

            def wait_g(k):
                gi, gb, gs = G[k]
                pltpu.make_async_copy(y.at[gi], gb, gs).wait()

            def fire_s(b, m):
                si, sb, ss = S[m]
                for t in range(BR):
                    si[pl.ds(t * 16, 16)] = gdst[pl.ds(b * BATCH + t * 16, 16)]
                pltpu.async_copy(sb, shared.at[si], ss, add=True)

            def wait_s(m):
                si, sb, ss = S[m]
                pltpu.make_async_copy(sb, shared.at[si], ss).wait()

            def scale(b, k, m):
                gb = G[k][1]
                sb = S[m][1]

                @pl.loop(0, BATCH, unroll=8)
                def sloop(j):
                    wv = plsc.load_gather(
                        gw, [jnp.full((16,), b * BATCH + j, jnp.int32)])
                    for t2 in range(4):
                        v = plsc.bitcast(
                            gb[j, pl.ds(t2 * 16, 16)], jnp.bfloat16)
                        a, c = plsc.unpack(
                            v, format=plsc.PackFormat.INTERLEAVED)
                        sb[j, pl.ds(t2 * 16, 16)] = a * wv
                        sb[j, pl.ds(64 + t2 * 16, 16)] = c * wv

            for j in range(4):
                @pl.when(j < nb)
                def _(j=j):
                    stage_fire(j, j)

            @pl.loop(0, (nb + 3) // 4)
            def ploop(i):
                for u in range(4):
                    b = 4 * i + u

                    @pl.when(b < nb)
                    def _(b=b, k=u, m=u % 2):
                        wait_g(k)

                        @pl.when(b >= 2)
                        def _():
                            wait_s(m)

                        scale(b, k, m)

                        @pl.when(b + 4 < nb)
                        def _():
                            stage_fire(b + 4, k)

                        fire_s(b, m)

            po = lax.rem(nb - 1, 2)

            @pl.when((nb >= 1) & (po == 0))
            def _():
                wait_s(0)

            @pl.when((nb >= 1) & (po == 1))
            def _():
                wait_s(1)

            @pl.when((nb >= 2) & (po == 1))
            def _():
                wait_s(0)

            @pl.when((nb >= 2) & (po == 0))
            def _():
                wait_s(1)

        plsc.subcore_barrier()
        pltpu.sync_copy(shared.at[pl.ds(sid * RPT, RPT)],
                        out.at[pl.ds(lo + sid * RPT, RPT)])


@functools.cache
def _get_sc_scatter():
    return pl.kernel(
        _sc_scatter_body,
        out_type=jax.ShapeDtypeStruct((PN, D), jnp.float32),
        mesh=plsc.VectorSubcoreMesh(core_axis_name="c", subcore_axis_name="s"),
        compiler_params=pltpu.CompilerParams(
            needs_layout_passes=False, use_tc_tiling_on_sc=False),
        scratch_types=[
        pltpu.VMEM((BLKE,), jnp.int32),
        pltpu.VMEM((BLKE,), jnp.int32),
        pltpu.VMEM((CAP,), jnp.int32),
        pltpu.VMEM((CAP,), jnp.int32),
        pltpu.VMEM((CAP,), jnp.float32),
        pltpu.VMEM((BATCH,), jnp.int32),
        pltpu.VMEM((BATCH,), jnp.int32),
        pltpu.VMEM((BATCH,), jnp.int32),
        pltpu.VMEM((BATCH,), jnp.int32),
        pltpu.VMEM((BATCH, D // 2), jnp.int32),
        pltpu.VMEM((BATCH, D // 2), jnp.int32),
        pltpu.VMEM((BATCH, D // 2), jnp.int32),
        pltpu.VMEM((BATCH, D // 2), jnp.int32),
        pltpu.VMEM((BATCH,), jnp.int32),
        pltpu.VMEM((BATCH,), jnp.int32),
        pltpu.VMEM((BATCH, D), jnp.float32),
        pltpu.VMEM((BATCH, D), jnp.float32),
            pltpu.VMEM((352,), jnp.float32),
            pltpu.VMEM_SHARED((CHUNK + 8, D), jnp.float32),
            pltpu.SemaphoreType.DMA,
            pltpu.SemaphoreType.DMA,
            pltpu.SemaphoreType.DMA,
            pltpu.SemaphoreType.DMA,
            pltpu.SemaphoreType.DMA,
            pltpu.SemaphoreType.DMA,
        ],
    )


@jax.jit
def kernel(x, edge_index, edge_weights, W_rel, b_rel, W_root):
    x_pad = jnp.zeros((PN, D), jnp.float32).at[:N_NODES].set(x)
    perm = jnp.stack(
        [jnp.arange(64, dtype=jnp.int32),
         jnp.arange(64, dtype=jnp.int32) + 64], axis=1).reshape(-1)
    y_pad, base_pad = _tc_matmuls(
        x_pad, W_rel.T[:, perm], W_root.T, b_rel.reshape(1, D))
    y32 = lax.bitcast_convert_type(
        y_pad.reshape(PN, D // 2, 2), jnp.int32)

    ei = edge_index.astype(jnp.int32)
    pad = PE - E_TOTAL
    ei_pad = jnp.concatenate(
        [ei,
         jnp.stack([jnp.zeros((pad,), jnp.int32),
                    jnp.full((pad,), 2 * N_NODES, jnp.int32)])],
        axis=1)

    w_pad = jnp.zeros((352,), jnp.float32).at[:N_EW].set(edge_weights)

    out_pad = _get_sc_scatter()(ei_pad, y32, base_pad, w_pad)
    return out_pad[:N_NODES]


# speedup vs baseline: 1.9080x; 1.0275x over previous
import functools

import jax
import jax.numpy as jnp
from jax import lax
from jax.experimental import pallas as pl
from jax.experimental.pallas import tpu as pltpu
from jax.experimental.pallas import tpu_sc as plsc

N_NODES = 34200
E_TOTAL = 615600
N_EW = 342
D = 128

PN = 34304
CHUNK = 8576
RPT = 536
DUMP = CHUNK
PE = 622592
TPT = 38912
BLKE = 2048
NBLK = TPT // BLKE
BATCH = 16
BR = BATCH // 16
CAP = BLKE + BATCH
NPASS = 2

TC_ROWS = 2144


def _tc_body(x_ref, wr_ref, wt_ref, b_ref, y_ref, base_ref):
    xb = x_ref[...]
    y_ref[...] = jnp.dot(
        xb, wr_ref[...], preferred_element_type=jnp.float32
    ).astype(jnp.bfloat16)
    base_ref[...] = (
        jnp.dot(xb, wt_ref[...], preferred_element_type=jnp.float32) + b_ref[...]
    )


def _tc_matmuls(x_pad, wr_t, wt_t, b2):
    grid = PN // TC_ROWS
    return pl.pallas_call(
        _tc_body,
        grid=(grid,),
        in_specs=[
            pl.BlockSpec((TC_ROWS, D), lambda i: (i, 0)),
            pl.BlockSpec((D, D), lambda i: (0, 0)),
            pl.BlockSpec((D, D), lambda i: (0, 0)),
            pl.BlockSpec((1, D), lambda i: (0, 0)),
        ],
        out_specs=[
            pl.BlockSpec((TC_ROWS, D), lambda i: (i, 0)),
            pl.BlockSpec((TC_ROWS, D), lambda i: (i, 0)),
        ],
        out_shape=[
            jax.ShapeDtypeStruct((PN, D), jnp.bfloat16),
            jax.ShapeDtypeStruct((PN, D), jnp.float32),
        ],
    )(x_pad, wr_t, wt_t, b2)


def _sc_scatter_body(ei, y, base, wts, out,
                     srcb, dstb, gsrc, gdst, gw,
                     gidx0, gidx1, gidx2, gidx3,
                     gbuf0, gbuf1, gbuf2, gbuf3,
                     sidx0, sidx1, sbuf0, sbuf1, wtab, shared,
                     gsem0, gsem1, gsem2, gsem3, ssem0, ssem1):
    cid = lax.axis_index("c")
    sid = lax.axis_index("s")
    iota16 = lax.iota(jnp.int32, 16)

    pltpu.sync_copy(wts, wtab)

    for p in range(NPASS):
        chunk_id = 2 * p + cid
        lo = chunk_id * CHUNK

        plsc.subcore_barrier()
        pltpu.sync_copy(base.at[pl.ds(lo + sid * RPT, RPT)],
                        shared.at[pl.ds(sid * RPT, RPT)])
        plsc.subcore_barrier()

        @pl.loop(0, NBLK)
        def blkloop(blk, lo=lo):
            ebase = sid * TPT + blk * BLKE
            pltpu.sync_copy(ei.at[0, pl.ds(ebase, BLKE)], srcb)
            pltpu.sync_copy(ei.at[1, pl.ds(ebase, BLKE)], dstb)

            @pl.loop(0, BLKE // 16, init_carry=jnp.int32(0), unroll=4)
            def cloop(j, off, ebase=ebase, lo=lo):
                s = srcb[pl.ds(j * 16, 16)]
                d = dstb[pl.ds(j * 16, 16)]
                m = (d >= lo) & (d < lo + CHUNK)
                rel = jnp.where(m, d - lo, DUMP)
                g0 = lax.rem(ebase + j * 16, N_EW)
                lane = g0 + iota16
                lane = jnp.where(lane >= N_EW, lane - N_EW, lane)
                w = plsc.load_gather(wtab, [lane])
                plsc.store_compressed(gsrc.at[pl.ds(off, 16)], s, mask=m)
                plsc.store_compressed(gdst.at[pl.ds(off, 16)], rel, mask=m)
                plsc.store_compressed(gw.at[pl.ds(off, 16)], w, mask=m)
                cnt = plsc.all_reduce_population_count(m)
                return off + jnp.max(cnt)

            n = cloop

            zi = jnp.zeros((16,), jnp.int32)
            dmp = jnp.full((16,), DUMP, jnp.int32)
            zf = jnp.zeros((16,), jnp.float32)
            for t in range(BR):
                gsrc[pl.ds(n + t * 16, 16)] = zi
                gdst[pl.ds(n + t * 16, 16)] = dmp
                gw[pl.ds(n + t * 16, 16)] = zf

            nb = (n + BATCH - 1) // BATCH

            G = ((gidx0, gbuf0, gsem0), (gidx1, gbuf1, gsem1),
                 (gidx2, gbuf2, gsem2), (gidx3, gbuf3, gsem3))
            S = ((sidx0, sbuf0, ssem0), (sidx1, sbuf1, ssem1))

            def stage_fire(b, k):
                gi, gb, gs = G[k]
                for t in range(BR):
                    gi[pl.ds(t * 16, 16)] = gsrc[pl.ds(b * BATCH + t * 16, 16)]
                pltpu.async_copy(y.at[gi], gb, gs)

            def wait_g(k):
                gi, gb, gs = G[k]
                pltpu.make_async_copy(y.at[gi], gb, gs).wait()

            def fire_s(b, m):
                si, sb, ss = S[m]
                for t in range(BR):
                    si[pl.ds(t * 16, 16)] = gdst[pl.ds(b * BATCH + t * 16, 16)]
                pltpu.async_copy(sb, shared.at[si], ss, add=True)

            def wait_s(m):
                si, sb, ss = S[m]
                pltpu.make_async_copy(sb, shared.at[si], ss).wait()

            def scale(b, k, m):
                gb = G[k][1]
                sb = S[m][1]

                @pl.loop(0, BATCH, unroll=8)
                def sloop(j):
                    wv = plsc.load_gather(
                        gw, [jnp.full((16,), b * BATCH + j, jnp.int32)])
                    for t2 in range(4):
                        v = plsc.bitcast(
                            gb[j, pl.ds(t2 * 16, 16)], jnp.bfloat16)
                        a, c = plsc.unpack(
                            v, format=plsc.PackFormat.INTERLEAVED)
                        sb[j, pl.ds(t2 * 16, 16)] = a * wv
                        sb[j, pl.ds(64 + t2 * 16, 16)] = c * wv

            for j in range(4):
                @pl.when(j < nb)
                def _(j=j):
                    stage_fire(j, j)

            @pl.loop(0, (nb + 3) // 4)
            def ploop(i):
                for u in range(4):
                    b = 4 * i + u

                    @pl.when(b < nb)
                    def _(b=b, k=u, m=u % 2):
                        wait_g(k)

                        @pl.when(b >= 2)
                        def _():
                            wait_s(m)

                        scale(b, k, m)

                        @pl.when(b + 4 < nb)
                        def _():
                            stage_fire(b + 4, k)

                        fire_s(b, m)

            po = lax.rem(nb - 1, 2)

            @pl.when((nb >= 1) & (po == 0))
            def _():
                wait_s(0)

            @pl.when((nb >= 1) & (po == 1))
            def _():
                wait_s(1)

            @pl.when((nb >= 2) & (po == 1))
            def _():
                wait_s(0)

            @pl.when((nb >= 2) & (po == 0))
            def _():
                wait_s(1)

        plsc.subcore_barrier()
        pltpu.sync_copy(shared.at[pl.ds(sid * RPT, RPT)],
                        out.at[pl.ds(lo + sid * RPT, RPT)])


@functools.cache
def _get_sc_scatter():
    return pl.kernel(
        _sc_scatter_body,
        out_type=jax.ShapeDtypeStruct((PN, D), jnp.float32),
        mesh=plsc.VectorSubcoreMesh(core_axis_name="c", subcore_axis_name="s"),
        compiler_params=pltpu.CompilerParams(
            needs_layout_passes=False, use_tc_tiling_on_sc=False),
        scratch_types=[
        pltpu.VMEM((BLKE,), jnp.int32),
        pltpu.VMEM((BLKE,), jnp.int32),
        pltpu.VMEM((CAP,), jnp.int32),
        pltpu.VMEM((CAP,), jnp.int32),
        pltpu.VMEM((CAP,), jnp.float32),
        pltpu.VMEM((BATCH,), jnp.int32),
        pltpu.VMEM((BATCH,), jnp.int32),
        pltpu.VMEM((BATCH,), jnp.int32),
        pltpu.VMEM((BATCH,), jnp.int32),
        pltpu.VMEM((BATCH, D // 2), jnp.int32),
        pltpu.VMEM((BATCH, D // 2), jnp.int32),
        pltpu.VMEM((BATCH, D // 2), jnp.int32),
        pltpu.VMEM((BATCH, D // 2), jnp.int32),
        pltpu.VMEM((BATCH,), jnp.int32),
        pltpu.VMEM((BATCH,), jnp.int32),
        pltpu.VMEM((BATCH, D), jnp.float32),
        pltpu.VMEM((BATCH, D), jnp.float32),
            pltpu.VMEM((352,), jnp.float32),
            pltpu.VMEM_SHARED((CHUNK + 8, D), jnp.float32),
            pltpu.SemaphoreType.DMA,
            pltpu.SemaphoreType.DMA,
            pltpu.SemaphoreType.DMA,
            pltpu.SemaphoreType.DMA,
            pltpu.SemaphoreType.DMA,
            pltpu.SemaphoreType.DMA,
        ],
    )


@jax.jit
def kernel(x, edge_index, edge_weights, W_rel, b_rel, W_root):
    x_pad = jnp.zeros((PN, D), jnp.float32).at[:N_NODES].set(x)
    perm = jnp.stack(
        [jnp.arange(64, dtype=jnp.int32),
         jnp.arange(64, dtype=jnp.int32) + 64], axis=1).reshape(-1)
    y_pad, base_pad = _tc_matmuls(
        x_pad, W_rel.T[:, perm], W_root.T, b_rel.reshape(1, D))
    y32 = lax.bitcast_convert_type(
        y_pad.reshape(PN, D // 2, 2), jnp.int32)

    ei = edge_index.astype(jnp.int32)
    pad = PE - E_TOTAL
    ei_pad = jnp.concatenate(
        [ei,
         jnp.stack([jnp.zeros((pad,), jnp.int32),
                    jnp.full((pad,), 2 * N_NODES, jnp.int32)])],
        axis=1)

    w_pad = jnp.zeros((352,), jnp.float32).at[:N_EW].set(edge_weights)

    out_pad = _get_sc_scatter()(ei_pad, y32, base_pad, w_pad)
    return out_pad[:N_NODES]
